# Initial kernel scaffold; baseline (speedup 1.0000x reference)
import jax
import jax.numpy as jnp
from jax.experimental import pallas as pl


def kernel(x, x_cov, adj_vals, feature_corr, bn_gamma, bn_beta, p0_W, p0_C, p1_W, p1_C, mlp_W1, mlp_b1, mlp_a1, mlp_W2, mlp_b2, mlp_a2, mlp_W3, mlp_b3, edge_index):
    raise NotImplementedError("write your pallas kernel here")


# speedup vs baseline: 2.4814x; 2.4814x over previous
import jax
import jax.numpy as jnp
from jax import lax
from jax.experimental import pallas as pl
from jax.experimental.pallas import tpu as pltpu
from jax.experimental.pallas import tpu_sc as plsc

N = 10000
NP = 10240
E = 160000
F = 128
FH = 64
C0 = 100
CP = 128
NCLASS = 40
NS = 16
NC = 2
ECB = 128
ECA = 64
NCHB = E // ECB
NCHA = E // ECA
RT = NP // NS

_f32 = jnp.float32
_i32 = jnp.int32


def _mesh():
    return plsc.VectorSubcoreMesh(core_axis_name="c", subcore_axis_name="s",
                                  num_cores=NC, num_subcores=NS)


def _prep_body(x_ref, xcov_ref, g_ref, b_ref, p0w_ref, p0c_ref, fc_ref,
               x0_ref, s0_ref, gp_ref, loss_ref):
    x = x_ref[...]
    mean = jnp.mean(x, axis=0, keepdims=True)
    var = jnp.mean((x - mean) ** 2, axis=0, keepdims=True)
    x0 = (x - mean) * lax.rsqrt(var + 1e-5) * g_ref[...] + b_ref[...]
    x0_ref[...] = x0

    xcov = xcov_ref[...]
    h0 = jnp.tanh(jnp.dot(xcov, p0w_ref[...], preferred_element_type=_f32))
    logits = lax.dot_general(h0, p0c_ref[...], (((1,), (1,)), ((), ())),
                             preferred_element_type=_f32)
    col = lax.broadcasted_iota(_i32, (N, CP), 1)
    lm = jnp.where(col < C0, logits, -1e30)
    m = jnp.max(lm, axis=1, keepdims=True)
    e = jnp.exp(lm - m)
    s0 = e / jnp.sum(e, axis=1, keepdims=True)
    s0_ref[...] = s0

    ones_n = jnp.ones((N, 1), _f32)
    denom = lax.dot_general(s0, ones_n, (((0,), (0,)), ((), ())),
                            preferred_element_type=_f32)
    sc_t = lax.dot_general(s0, xcov, (((0,), (0,)), ((), ())),
                           preferred_element_type=_f32)
    xc1c = sc_t * (1.0 / (denom + 1e-8))
    xcov2 = jnp.sum(xc1c, axis=0, keepdims=True) / (C0 + 1e-8)
    row = lax.broadcasted_iota(_i32, (CP, F), 0)
    corr1 = jnp.abs(xc1c - xcov2)
    loss_ref[...] = jnp.sum(jnp.where(row < C0, corr1, 0.0),
                            keepdims=True).reshape(1, 1) / (C0 * F)
    z = fc_ref[...] - corr1
    gain = 1.0 + jnp.tanh(0.5 * z)
    g2 = gain * gain
    gp_ref[...] = jnp.where(row < C0, g2, 0.0)


def _tc_prep(x, x_cov, bn_gamma, bn_beta, p0_W, p0_Cp, fc):
    return pl.pallas_call(
        _prep_body,
        out_shape=(
            jax.ShapeDtypeStruct((N, F), _f32),
            jax.ShapeDtypeStruct((N, CP), _f32),
            jax.ShapeDtypeStruct((CP, F), _f32),
            jax.ShapeDtypeStruct((1, 1), _f32),
        ),
    )(x, x_cov, bn_gamma, bn_beta, p0_W, p0_Cp, fc)


def _edges_body(s0_hbm, edge_hbm, p_hbm, s0_sp, si, di, ga, gb, sem1, sem2):
    cid = lax.axis_index("c")
    sid = lax.axis_index("s")
    r0 = sid * RT
    for b in range(RT // ECB):
        rr = r0 + b * ECB
        pltpu.sync_copy(s0_hbm.at[pl.ds(rr, ECB)], ga)
        pltpu.sync_copy(ga, s0_sp.at[pl.ds(rr, ECB)])
    plsc.subcore_barrier()

    base = cid * (NCHB // NC)
    hi = base + (NCHB // NC)

    def step(k, carry):
        g = base + sid + NS * k

        @pl.when(g < hi)
        def _():
            e0 = g * ECB
            pltpu.sync_copy(edge_hbm.at[0, pl.ds(e0, ECB)], si)
            pltpu.sync_copy(edge_hbm.at[1, pl.ds(e0, ECB)], di)
            c1 = pltpu.async_copy(s0_sp.at[si], ga, sem1)
            c2 = pltpu.async_copy(s0_sp.at[di], gb, sem2)
            c1.wait()
            c2.wait()

            @plsc.parallel_loop(0, ECB)
            def _mul(r):
                for j in range(CP // 16):
                    sl = pl.ds(j * 16, 16)
                    ga[r, sl] = ga[r, sl] * gb[r, sl]

            pltpu.sync_copy(ga, p_hbm.at[pl.ds(e0, ECB)])

        return carry

    lax.fori_loop(0, (NCHB // NC + NS - 1) // NS, step, 0)


def _sc_edges(s0p, edge_index):
    f = pl.kernel(
        _edges_body,
        out_type=jax.ShapeDtypeStruct((E, CP), _f32),
        mesh=_mesh(),
        scratch_types=[
            pltpu.VMEM_SHARED((NP, CP), _f32),
            pltpu.VMEM((ECB,), _i32),
            pltpu.VMEM((ECB,), _i32),
            pltpu.VMEM((ECB, CP), _f32),
            pltpu.VMEM((ECB, CP), _f32),
            pltpu.SemaphoreType.DMA,
            pltpu.SemaphoreType.DMA,
        ],
    )
    return f(s0p, edge_index)


def _exp_body(p_ref, gp_ref, adj_ref, exa_ref, exb_ref):
    v = jnp.dot(p_ref[...], gp_ref[...], preferred_element_type=_f32)
    e = jnp.exp(v + adj_ref[...])
    exa_ref[...] = e[:, :FH]
    exb_ref[...] = e[:, FH:]


def _tc_exp(p, gp, adj2d):
    blk = 1280
    return pl.pallas_call(
        _exp_body,
        grid=(E // blk,),
        in_specs=[
            pl.BlockSpec((blk, CP), lambda i: (i, 0)),
            pl.BlockSpec((CP, F), lambda i: (0, 0)),
            pl.BlockSpec((blk, 1), lambda i: (i, 0)),
        ],
        out_specs=(
            pl.BlockSpec((blk, FH), lambda i: (i, 0)),
            pl.BlockSpec((blk, FH), lambda i: (i, 0)),
        ),
        out_shape=(
            jax.ShapeDtypeStruct((E, FH), _f32),
            jax.ShapeDtypeStruct((E, FH), _f32),
        ),
    )(p, gp, adj2d)


def _agg1_body(x0_hbm, edge_hbm, exa_hbm, exb_hbm,
               xc1a_hbm, xc1b_hbm, ra_hbm, rb_hbm,
               acc_sp, exv, gx, mg, xo, si, di, sem):
    cid = lax.axis_index("c")
    sid = lax.axis_index("s")
    r0 = sid * RT

    @plsc.parallel_loop(0, ECA)
    def _z(r):
        for j in range(F // 16):
            mg[r, pl.ds(j * 16, 16)] = jnp.zeros((16,), _f32)

    for b in range(RT // ECA):
        pltpu.sync_copy(mg, acc_sp.at[pl.ds(r0 + b * ECA, ECA)])
    plsc.subcore_barrier()

    def step(k, carry):
        g = sid + NS * k

        @pl.when(g < NCHA)
        def _():
            e0 = g * ECA
            pltpu.sync_copy(edge_hbm.at[0, pl.ds(e0, ECA)], si)
            pltpu.sync_copy(edge_hbm.at[1, pl.ds(e0, ECA)], di)
            pltpu.async_copy(x0_hbm.at[di], gx, sem).wait()

            @pl.when(cid == 0)
            def _():
                pltpu.sync_copy(exa_hbm.at[pl.ds(e0, ECA)], exv)

                @plsc.parallel_loop(0, ECA)
                def _mul(r):
                    for j in range(FH // 16):
                        sl = pl.ds(j * 16, 16)
                        v = exv[r, sl]
                        mg[r, sl] = v
                        mg[r, pl.ds(FH + j * 16, 16)] = v * gx[r, sl]

            @pl.when(cid == 1)
            def _():
                pltpu.sync_copy(exb_hbm.at[pl.ds(e0, ECA)], exv)

                @plsc.parallel_loop(0, ECA)
                def _mul(r):
                    for j in range(FH // 16):
                        sl = pl.ds(j * 16, 16)
                        slh = pl.ds(FH + j * 16, 16)
                        v = exv[r, sl]
                        mg[r, sl] = v
                        mg[r, slh] = v * gx[r, slh]

            pltpu.sync_copy(mg, acc_sp.at[si], add=True)

        return carry

    lax.fori_loop(0, (NCHA + NS - 1) // NS, step, 0)
    plsc.subcore_barrier()

    for b in range(RT // ECA):
        rr = r0 + b * ECA
        pltpu.sync_copy(acc_sp.at[pl.ds(rr, ECA)], mg)
        pltpu.sync_copy(x0_hbm.at[pl.ds(rr, ECA)], gx)

        @pl.when(cid == 0)
        def _():
            @plsc.parallel_loop(0, ECA)
            def _fin(r):
                for j in range(FH // 16):
                    sl = pl.ds(j * 16, 16)
                    rcp = 1.0 / (mg[r, sl] + 1e-16)
                    exv[r, sl] = rcp
                    xo[r, sl] = rcp * mg[r, pl.ds(FH + j * 16, 16)] + gx[r, sl]

            pltpu.sync_copy(exv, ra_hbm.at[pl.ds(rr, ECA)])
            pltpu.sync_copy(xo, xc1a_hbm.at[pl.ds(rr, ECA)])

        @pl.when(cid == 1)
        def _():
            @plsc.parallel_loop(0, ECA)
            def _fin(r):
                for j in range(FH // 16):
                    sl = pl.ds(j * 16, 16)
                    slh = pl.ds(FH + j * 16, 16)
                    rcp = 1.0 / (mg[r, sl] + 1e-16)
                    exv[r, sl] = rcp
                    xo[r, sl] = rcp * mg[r, slh] + gx[r, slh]

            pltpu.sync_copy(exv, rb_hbm.at[pl.ds(rr, ECA)])
            pltpu.sync_copy(xo, xc1b_hbm.at[pl.ds(rr, ECA)])


def _sc_agg1(x0_pad, edge_index, exa, exb):
    f = pl.kernel(
        _agg1_body,
        out_type=(
            jax.ShapeDtypeStruct((NP, FH), _f32),
            jax.ShapeDtypeStruct((NP, FH), _f32),
            jax.ShapeDtypeStruct((NP, FH), _f32),
            jax.ShapeDtypeStruct((NP, FH), _f32),
        ),
        mesh=_mesh(),
        scratch_types=[
            pltpu.VMEM_SHARED((NP, F), _f32),
            pltpu.VMEM((ECA, FH), _f32),
            pltpu.VMEM((ECA, F), _f32),
            pltpu.VMEM((ECA, F), _f32),
            pltpu.VMEM((ECA, FH), _f32),
            pltpu.VMEM((ECA,), _i32),
            pltpu.VMEM((ECA,), _i32),
            pltpu.SemaphoreType.DMA,
        ],
    )
    return f(x0_pad, edge_index, exa, exb)


def _agg2_body(x0_hbm, xc1_hbm, edge_hbm, exa_hbm, exb_hbm, ra_hbm, rb_hbm,
               xc2a_hbm, xc2b_hbm,
               acc_sp, exv, gx, mg, xo, si, di, sem):
    cid = lax.axis_index("c")
    sid = lax.axis_index("s")
    r0 = sid * RT

    @plsc.parallel_loop(0, ECA)
    def _z(r):
        for j in range(F // 16):
            mg[r, pl.ds(j * 16, 16)] = jnp.zeros((16,), _f32)

    for b in range(RT // ECA):
        pltpu.sync_copy(mg, acc_sp.at[pl.ds(r0 + b * ECA, ECA)])
    plsc.subcore_barrier()

    def step(k, carry):
        g = sid + NS * k

        @pl.when(g < NCHA)
        def _():
            e0 = g * ECA
            pltpu.sync_copy(edge_hbm.at[0, pl.ds(e0, ECA)], si)
            pltpu.sync_copy(edge_hbm.at[1, pl.ds(e0, ECA)], di)
            pltpu.async_copy(xc1_hbm.at[di], gx, sem).wait()

            @pl.when(cid == 0)
            def _():
                pltpu.sync_copy(exa_hbm.at[pl.ds(e0, ECA)], exv)

                @plsc.parallel_loop(0, ECA)
                def _mul(r):
                    for j in range(FH // 16):
                        sl = pl.ds(j * 16, 16)
                        mg[r, sl] = exv[r, sl] * gx[r, sl]

            @pl.when(cid == 1)
            def _():
                pltpu.sync_copy(exb_hbm.at[pl.ds(e0, ECA)], exv)

                @plsc.parallel_loop(0, ECA)
                def _mul(r):
                    for j in range(FH // 16):
                        sl = pl.ds(j * 16, 16)
                        mg[r, sl] = exv[r, sl] * gx[r, pl.ds(FH + j * 16, 16)]

            pltpu.sync_copy(mg, acc_sp.at[si], add=True)

        return carry

    lax.fori_loop(0, (NCHA + NS - 1) // NS, step, 0)
    plsc.subcore_barrier()

    for b in range(RT // ECA):
        rr = r0 + b * ECA
        pltpu.sync_copy(acc_sp.at[pl.ds(rr, ECA)], mg)
        pltpu.sync_copy(x0_hbm.at[pl.ds(rr, ECA)], gx)

        @pl.when(cid == 0)
        def _():
            pltpu.sync_copy(ra_hbm.at[pl.ds(rr, ECA)], exv)

            @plsc.parallel_loop(0, ECA)
            def _fin(r):
                for j in range(FH // 16):
                    sl = pl.ds(j * 16, 16)
                    xo[r, sl] = exv[r, sl] * mg[r, sl] + gx[r, sl]

            pltpu.sync_copy(xo, xc2a_hbm.at[pl.ds(rr, ECA)])

        @pl.when(cid == 1)
        def _():
            pltpu.sync_copy(rb_hbm.at[pl.ds(rr, ECA)], exv)

            @plsc.parallel_loop(0, ECA)
            def _fin(r):
                for j in range(FH // 16):
                    sl = pl.ds(j * 16, 16)
                    xo[r, sl] = (exv[r, sl] * mg[r, sl]
                                 + gx[r, pl.ds(FH + j * 16, 16)])

            pltpu.sync_copy(xo, xc2b_hbm.at[pl.ds(rr, ECA)])


def _sc_agg2(x0_pad, xc1_full, edge_index, exa, exb, ra, rb):
    f = pl.kernel(
        _agg2_body,
        out_type=(
            jax.ShapeDtypeStruct((NP, FH), _f32),
            jax.ShapeDtypeStruct((NP, FH), _f32),
        ),
        mesh=_mesh(),
        scratch_types=[
            pltpu.VMEM_SHARED((NP, F), _f32),
            pltpu.VMEM((ECA, FH), _f32),
            pltpu.VMEM((ECA, F), _f32),
            pltpu.VMEM((ECA, F), _f32),
            pltpu.VMEM((ECA, FH), _f32),
            pltpu.VMEM((ECA,), _i32),
            pltpu.VMEM((ECA,), _i32),
            pltpu.SemaphoreType.DMA,
        ],
    )
    return f(x0_pad, xc1_full, edge_index, exa, exb, ra, rb)


def _mlp_body(xc2a_ref, xc2b_ref, x0_ref, w1_ref, b1_ref, a1_ref, w2_ref,
              b2_ref, a2_ref, w3_ref, b3_ref, out_ref):
    x0 = x0_ref[...]
    w1 = w1_ref[...]
    h = (jnp.dot(xc2a_ref[...], w1[:FH, :], preferred_element_type=_f32)
         + jnp.dot(xc2b_ref[...], w1[FH:F, :], preferred_element_type=_f32)
         + jnp.dot(x0, w1[F:, :], preferred_element_type=_f32)
         + b1_ref[...])
    a1 = a1_ref[...]
    h = jnp.where(h >= 0, h, a1 * h)
    h = jnp.dot(h, w2_ref[...], preferred_element_type=_f32) + b2_ref[...]
    a2 = a2_ref[...]
    h = jnp.where(h >= 0, h, a2 * h)
    lg = jnp.dot(h, w3_ref[...], preferred_element_type=_f32) + b3_ref[...]
    m = jnp.max(lg, axis=1, keepdims=True)
    lse = jnp.log(jnp.sum(jnp.exp(lg - m), axis=1, keepdims=True))
    out_ref[...] = lg - m - lse


def _tc_mlp(xc2a, xc2b, x0, w1, b1, a1, w2, b2, a2, w3, b3):
    return pl.pallas_call(
        _mlp_body,
        out_shape=jax.ShapeDtypeStruct((N, NCLASS), _f32),
    )(xc2a, xc2b, x0, w1, b1, a1, w2, b2, a2, w3, b3)


def kernel(x, x_cov, adj_vals, feature_corr, bn_gamma, bn_beta, p0_W, p0_C,
           p1_W, p1_C, mlp_W1, mlp_b1, mlp_a1, mlp_W2, mlp_b2, mlp_a2,
           mlp_W3, mlp_b3, edge_index):
    edge_index = edge_index.astype(_i32)
    p0_Cp = jnp.pad(p0_C, ((0, CP - C0), (0, 0)))
    x0, s0p, gp, loss = _tc_prep(
        x, x_cov, bn_gamma.reshape(1, F), bn_beta.reshape(1, F), p0_W, p0_Cp,
        feature_corr.reshape(1, F))
    s0p_pad = jnp.pad(s0p, ((0, NP - N), (0, 0)))
    x0_pad = jnp.pad(x0, ((0, NP - N), (0, 0)))
    p = _sc_edges(s0p_pad, edge_index)
    exa, exb = _tc_exp(p, gp, adj_vals[:, None])
    xc1a, xc1b, ra, rb = _sc_agg1(x0_pad, edge_index, exa, exb)
    xc1_full = jnp.concatenate([xc1a, xc1b], axis=1)
    xc2a, xc2b = _sc_agg2(x0_pad, xc1_full, edge_index, exa, exb, ra, rb)
    out = _tc_mlp(xc2a[:N], xc2b[:N], x0, mlp_W1,
                  mlp_b1.reshape(1, F), mlp_a1.reshape(1, 1), mlp_W2,
                  mlp_b2.reshape(1, F), mlp_a2.reshape(1, 1), mlp_W3,
                  mlp_b3.reshape(1, NCLASS))
    return (out, loss.reshape(()))
